# baseline (speedup 1.0000x reference)
import functools

import jax
import jax.numpy as jnp
import numpy as np
from jax import lax
from jax.experimental import pallas as pl
from jax.experimental.pallas import tpu as pltpu
from jax.experimental.pallas import tpu_sc as plsc

MAXLEN = 8192
EMBED_DIM = 64
BATCH = 4
SEQ = 4096

B = BATCH * SEQ
NW = 32
B_PER_W = B // NW
CHUNK = 128
NCHUNK = B_PER_W // CHUNK


def _positional_encoding_np(position, d_model):
    pos = np.arange(position)[:, np.newaxis].astype(np.float64)
    i = np.arange(d_model)[np.newaxis, :].astype(np.float64)
    angle_rates = 1.0 / np.power(10000, 2 * (i // 2) / np.float32(d_model))
    angle_rads = pos * angle_rates
    angle_rads[:, 0::2] = np.sin(angle_rads[:, 0::2])
    angle_rads[:, 1::2] = np.cos(angle_rads[:, 1::2])
    return angle_rads.astype(np.float32)


_POS_NP = _positional_encoding_np(SEQ, EMBED_DIM)


def _embed_body(table_hbm, idx_hbm, pos_hbm, out_hbm, idx_v, rows_v, sem):
    wid = lax.axis_index("s") * 2 + lax.axis_index("c")
    base = wid * B_PER_W
    pbase = (wid % (SEQ // B_PER_W)) * B_PER_W
    pltpu.sync_copy(idx_hbm.at[pl.ds(wid * NCHUNK, NCHUNK)], idx_v)
    pltpu.sync_copy(pos_hbm.at[pl.ds(pbase, B_PER_W)], rows_v)
    copies = [
        pltpu.async_copy(
            table_hbm.at[idx_v.at[c]],
            rows_v.at[pl.ds(c * CHUNK, CHUNK)],
            sem,
            add=True,
        )
        for c in range(NCHUNK)
    ]
    for cp in copies:
        cp.wait()
    pltpu.sync_copy(rows_v, out_hbm.at[pl.ds(base, B_PER_W)])


_mesh = plsc.VectorSubcoreMesh(core_axis_name="c", subcore_axis_name="s")

_embed = pl.kernel(
    _embed_body,
    out_type=jax.ShapeDtypeStruct((B, EMBED_DIM), jnp.float32),
    mesh=_mesh,
    scratch_types=[
        pltpu.VMEM((NCHUNK, CHUNK), jnp.int32),
        pltpu.VMEM((B_PER_W, EMBED_DIM), jnp.float32),
        pltpu.SemaphoreType.DMA,
    ],
    compiler_params=pltpu.CompilerParams(use_tc_tiling_on_sc=False),
)


@jax.jit
def kernel(x, table):
    idx2d = x.reshape(NW * NCHUNK, CHUNK).astype(jnp.int32)
    out = _embed(table, idx2d, jnp.asarray(_POS_NP))
    return out.reshape(BATCH, SEQ, EMBED_DIM)
